# baseline (speedup 1.0000x reference)
import functools

import jax
import jax.numpy as jnp
from jax import lax
from jax.experimental import pallas as pl
from jax.experimental.pallas import tpu as pltpu
from jax.experimental.pallas import tpu_sc as plsc

_N = 50000
_NP = 50176
_S = 1600000
_T = 512
_R = 5
_NW = 32
_SYN_W = _S // _NW
_CHUNK = 400
_NCHUNK = _SYN_W // _CHUNK
_UNROLL = 5
_VSTEPS = _CHUNK // 16 // _UNROLL
_NS = _NP // _NW
_NS_LAST = _N - (_NW - 1) * _NS

_mesh = plsc.VectorSubcoreMesh(core_axis_name="c", subcore_axis_name="s")
_cp = pltpu.CompilerParams(needs_layout_passes=False, use_tc_tiling_on_sc=False)


@functools.partial(
    pl.kernel,
    out_type=jax.ShapeDtypeStruct((_NW * _NP,), jnp.float32),
    mesh=_mesh,
    compiler_params=_cp,
    scratch_types=[
        pltpu.VMEM((_N,), jnp.float32),
        pltpu.VMEM((_T * _R,), jnp.float32),
        pltpu.VMEM((_T,), jnp.float32),
        pltpu.VMEM((_NP,), jnp.float32),
        pltpu.VMEM((_CHUNK, 2), jnp.int32),
        pltpu.VMEM((_CHUNK, 2), jnp.int32),
        pltpu.VMEM((_CHUNK,), jnp.int32),
        pltpu.VMEM((_CHUNK,), jnp.int32),
        pltpu.VMEM((_CHUNK,), jnp.float32),
        pltpu.VMEM((_CHUNK,), jnp.float32),
        pltpu.SemaphoreType.DMA,
        pltpu.SemaphoreType.DMA,
        pltpu.SemaphoreType.DMA,
        pltpu.SemaphoreType.DMA,
    ],
)
def _sc_synapse_kernel(idx_hbm, sid_hbm, w_hbm, spikes_hbm,
                       basis_hbm, out_hbm, spikes_v, basis_v, bsum_v,
                       acc_v, idx0_v, idx1_v,
                       sid0_v, sid1_v, w0_v, w1_v, sem0, sem1, semt, semb):
    cid = lax.axis_index("c")
    sub = lax.axis_index("s")
    wid = cid * 16 + sub
    base = wid * _SYN_W

    idx_bufs = (idx0_v, idx1_v)
    sid_bufs = (sid0_v, sid1_v)
    w_bufs = (w0_v, w1_v)
    sems = (sem0, sem1)

    h_sp = pltpu.async_copy(spikes_hbm, spikes_v, semt)
    h_ba = pltpu.async_copy(basis_hbm, basis_v, semb)

    def issue(c, slot):
        off = base + c * _CHUNK
        pltpu.async_copy(idx_hbm.at[pl.ds(off, _CHUNK), :],
                         idx_bufs[slot], sems[slot])
        pltpu.async_copy(sid_hbm.at[pl.ds(off, _CHUNK)],
                         sid_bufs[slot], sems[slot])
        pltpu.async_copy(w_hbm.at[pl.ds(off, _CHUNK)],
                         w_bufs[slot], sems[slot])

    def drain(slot):
        pltpu.make_async_copy(idx_hbm.at[pl.ds(0, _CHUNK), :],
                              idx_bufs[slot], sems[slot]).wait()
        pltpu.make_async_copy(sid_hbm.at[pl.ds(0, _CHUNK)],
                              sid_bufs[slot], sems[slot]).wait()
        pltpu.make_async_copy(w_hbm.at[pl.ds(0, _CHUNK)],
                              w_bufs[slot], sems[slot]).wait()

    issue(0, 0)

    zeros = jnp.zeros((16,), jnp.float32)

    def zero_body(i, c):
        for k in range(4):
            acc_v[pl.ds((i * 4 + k) * 16, 16)] = zeros
        return c

    lax.fori_loop(0, _NP // 16 // 4, zero_body, 0)

    h_ba.wait()
    lane = jnp.arange(16, dtype=jnp.int32)

    def bsum_body(i, c):
        t5 = (lane + i * 16) * _R
        s = plsc.load_gather(basis_v, [t5])
        for r in range(1, _R):
            s = s + plsc.load_gather(basis_v, [t5 + r])
        bsum_v[pl.ds(i * 16, 16)] = s
        return c

    lax.fori_loop(0, _T // 16, bsum_body, 0)
    h_sp.wait()

    col0 = jnp.zeros((16,), jnp.int32)
    col1 = jnp.ones((16,), jnp.int32)

    def compute(slot):
        idx_v = idx_bufs[slot]
        sid_v = sid_bufs[slot]
        w_v = w_bufs[slot]

        def body(i, cc):
            for k in range(_UNROLL):
                j = i * _UNROLL + k
                s_vec = lane + j * 16
                q = plsc.load_gather(idx_v, [s_vec, col0])
                p = plsc.load_gather(idx_v, [s_vec, col1])
                t = sid_v[pl.ds(j * 16, 16)]
                w = w_v[pl.ds(j * 16, 16)]
                z = plsc.load_gather(spikes_v, [p])
                b = plsc.load_gather(bsum_v, [t])
                vv = z * w * b
                plsc.addupdate_scatter(acc_v, [q], vv, mask=z > 0.0)
            return cc

        lax.fori_loop(0, _VSTEPS, body, 0)

    def pair_body(g, carry):
        c = g * 2
        issue(c + 1, 1)
        drain(0)
        compute(0)
        issue(c + 2, 0)
        drain(1)
        compute(1)
        return carry

    lax.fori_loop(0, (_NCHUNK - 1) // 2, pair_body, 0)
    drain(0)
    compute(0)

    pltpu.sync_copy(acc_v, out_hbm.at[pl.ds(wid * _NP, _NP)])


@functools.partial(
    pl.kernel,
    out_type=jax.ShapeDtypeStruct((_NP,), jnp.float32),
    mesh=_mesh,
    compiler_params=_cp,
    scratch_types=[
        pltpu.VMEM((_NW * _NS,), jnp.float32),
        pltpu.VMEM((_NS,), jnp.float32),
        pltpu.VMEM((_NS,), jnp.float32),
        pltpu.VMEM((_NS,), jnp.float32),
        pltpu.VMEM((_NS,), jnp.float32),
        pltpu.VMEM((_NS,), jnp.float32),
        pltpu.VMEM((_NS,), jnp.float32),
        pltpu.VMEM((_NS,), jnp.float32),
        pltpu.SemaphoreType.DMA,
        pltpu.SemaphoreType.DMA,
    ],
)
def _sc_membrane_kernel(part_hbm, v_hbm, sp_hbm, dec_hbm, cf_hbm, vth_hbm,
                        nrm_hbm, out_hbm, part_v, v_v, sp_v, dec_v, cf_v,
                        vth_v, nrm_v, out_v, semp, semn):
    cid = lax.axis_index("c")
    sub = lax.axis_index("s")
    wid = cid * 16 + sub
    toff = wid * _NS
    is_last = wid == _NW - 1

    hs = [pltpu.async_copy(part_hbm.at[pl.ds(k * _NP + toff, _NS)],
                           part_v.at[pl.ds(k * _NS, _NS)], semp)
          for k in range(_NW)]

    def issue_neuron(sz):
        return (
            pltpu.async_copy(v_hbm.at[pl.ds(toff, sz)],
                             v_v.at[pl.ds(0, sz)], semn),
            pltpu.async_copy(sp_hbm.at[pl.ds(toff, sz)],
                             sp_v.at[pl.ds(0, sz)], semn),
            pltpu.async_copy(dec_hbm.at[pl.ds(toff, sz)],
                             dec_v.at[pl.ds(0, sz)], semn),
            pltpu.async_copy(cf_hbm.at[pl.ds(toff, sz)],
                             cf_v.at[pl.ds(0, sz)], semn),
            pltpu.async_copy(vth_hbm.at[pl.ds(toff, sz)],
                             vth_v.at[pl.ds(0, sz)], semn),
            pltpu.async_copy(nrm_hbm.at[pl.ds(toff, sz)],
                             nrm_v.at[pl.ds(0, sz)], semn),
        )

    @pl.when(is_last)
    def _():
        for h in issue_neuron(_NS_LAST):
            h.wait()

    @pl.when(jnp.logical_not(is_last))
    def _():
        for h in issue_neuron(_NS):
            h.wait()

    for h in hs:
        h.wait()

    nvregs = jnp.where(is_last, _NS_LAST // 16, _NS // 16)

    def body(j, cc):
        sl = pl.ds(j * 16, 16)
        rec = part_v[pl.ds(j * 16, 16)]
        for k in range(1, _NW):
            rec = rec + part_v[pl.ds(k * _NS + j * 16, 16)]
        sp = sp_v[sl]
        v_reset = v_v[sl] * (1.0 - sp)
        new_v = dec_v[sl] * v_reset + cf_v[sl] * rec
        v_scaled = (new_v - vth_v[sl]) / nrm_v[sl]
        out_v[sl] = jnp.where(v_scaled > 0.0,
                              jnp.float32(1.0), jnp.float32(0.0))
        return cc

    lax.fori_loop(0, nvregs, body, 0)

    @pl.when(is_last)
    def _():
        pltpu.sync_copy(out_v.at[pl.ds(0, _NS_LAST)],
                        out_hbm.at[pl.ds(toff, _NS_LAST)])

    @pl.when(jnp.logical_not(is_last))
    def _():
        pltpu.sync_copy(out_v, out_hbm.at[pl.ds(toff, _NS)])


def kernel(spikes, v, weights, syn_ids, indices, basis, decay,
           current_factor, v_th, normalizer):
    basis_flat = basis.reshape(_T * _R)
    spikes_flat = spikes.reshape(_N)

    partial = _sc_synapse_kernel(indices, syn_ids, weights, spikes_flat,
                                 basis_flat)

    z = _sc_membrane_kernel(partial, v.reshape(_N), spikes_flat, decay,
                            current_factor, v_th, normalizer)
    return z[:_N].reshape(1, _N)


# speedup vs baseline: 17.4026x; 17.4026x over previous
import functools

import jax
import jax.numpy as jnp
from jax import lax
from jax.experimental import pallas as pl
from jax.experimental.pallas import tpu as pltpu
from jax.experimental.pallas import tpu_sc as plsc

_N = 50000
_NP = 50176
_S = 1600000
_T = 512
_R = 5
_NW = 32
_SYN_W = _S // _NW
_CHUNK = 2000
_NCHUNK = _SYN_W // _CHUNK
_UNROLL = 5
_VSTEPS = _CHUNK // 16 // _UNROLL
_NS = _NP // _NW
_NS_LAST = _N - (_NW - 1) * _NS

_mesh = plsc.VectorSubcoreMesh(core_axis_name="c", subcore_axis_name="s")
_cp = pltpu.CompilerParams(needs_layout_passes=False)


@functools.partial(
    pl.kernel,
    out_type=jax.ShapeDtypeStruct((_NW * _NP,), jnp.float32),
    mesh=_mesh,
    compiler_params=_cp,
    scratch_types=[
        pltpu.VMEM((_N,), jnp.float32),
        pltpu.VMEM((_T * _R,), jnp.float32),
        pltpu.VMEM((_T,), jnp.float32),
        pltpu.VMEM((_NP,), jnp.float32),
        pltpu.VMEM((_CHUNK,), jnp.int32),
        pltpu.VMEM((_CHUNK,), jnp.int32),
        pltpu.VMEM((_CHUNK,), jnp.int32),
        pltpu.VMEM((_CHUNK,), jnp.int32),
        pltpu.VMEM((_CHUNK,), jnp.int32),
        pltpu.VMEM((_CHUNK,), jnp.int32),
        pltpu.VMEM((_CHUNK,), jnp.float32),
        pltpu.VMEM((_CHUNK,), jnp.float32),
        pltpu.SemaphoreType.DMA,
        pltpu.SemaphoreType.DMA,
        pltpu.SemaphoreType.DMA,
        pltpu.SemaphoreType.DMA,
    ],
)
def _sc_synapse_kernel(pre_hbm, post_hbm, sid_hbm, w_hbm, spikes_hbm,
                       basis_hbm, out_hbm, spikes_v, basis_v, bsum_v,
                       acc_v, pre0_v, pre1_v, post0_v, post1_v,
                       sid0_v, sid1_v, w0_v, w1_v, sem0, sem1, semt, semb):
    cid = lax.axis_index("c")
    sub = lax.axis_index("s")
    wid = cid * 16 + sub
    base = wid * _SYN_W

    pre_bufs = (pre0_v, pre1_v)
    post_bufs = (post0_v, post1_v)
    sid_bufs = (sid0_v, sid1_v)
    w_bufs = (w0_v, w1_v)
    sems = (sem0, sem1)

    h_sp = pltpu.async_copy(spikes_hbm, spikes_v, semt)
    h_ba = pltpu.async_copy(basis_hbm, basis_v, semb)

    def issue(c, slot):
        off = base + c * _CHUNK
        pltpu.async_copy(pre_hbm.at[pl.ds(off, _CHUNK)],
                         pre_bufs[slot], sems[slot])
        pltpu.async_copy(post_hbm.at[pl.ds(off, _CHUNK)],
                         post_bufs[slot], sems[slot])
        pltpu.async_copy(sid_hbm.at[pl.ds(off, _CHUNK)],
                         sid_bufs[slot], sems[slot])
        pltpu.async_copy(w_hbm.at[pl.ds(off, _CHUNK)],
                         w_bufs[slot], sems[slot])

    def drain(slot):
        pltpu.make_async_copy(pre_hbm.at[pl.ds(0, _CHUNK)],
                              pre_bufs[slot], sems[slot]).wait()
        pltpu.make_async_copy(post_hbm.at[pl.ds(0, _CHUNK)],
                              post_bufs[slot], sems[slot]).wait()
        pltpu.make_async_copy(sid_hbm.at[pl.ds(0, _CHUNK)],
                              sid_bufs[slot], sems[slot]).wait()
        pltpu.make_async_copy(w_hbm.at[pl.ds(0, _CHUNK)],
                              w_bufs[slot], sems[slot]).wait()

    issue(0, 0)

    zeros = jnp.zeros((16,), jnp.float32)

    def zero_body(i, c):
        for k in range(4):
            acc_v[pl.ds((i * 4 + k) * 16, 16)] = zeros
        return c

    lax.fori_loop(0, _NP // 16 // 4, zero_body, 0)

    h_ba.wait()
    lane = jnp.arange(16, dtype=jnp.int32)

    def bsum_body(i, c):
        t5 = (lane + i * 16) * _R
        s = plsc.load_gather(basis_v, [t5])
        for r in range(1, _R):
            s = s + plsc.load_gather(basis_v, [t5 + r])
        bsum_v[pl.ds(i * 16, 16)] = s
        return c

    lax.fori_loop(0, _T // 16, bsum_body, 0)
    h_sp.wait()

    def compute(slot):
        pre_v = pre_bufs[slot]
        post_v = post_bufs[slot]
        sid_v = sid_bufs[slot]
        w_v = w_bufs[slot]

        def body(i, cc):
            for k in range(_UNROLL):
                j = i * _UNROLL + k
                p = pre_v[pl.ds(j * 16, 16)]
                q = post_v[pl.ds(j * 16, 16)]
                t = sid_v[pl.ds(j * 16, 16)]
                w = w_v[pl.ds(j * 16, 16)]
                z = plsc.load_gather(spikes_v, [p])
                b = plsc.load_gather(bsum_v, [t])
                vv = z * w * b
                plsc.addupdate_scatter(acc_v, [q], vv, mask=z > 0.0)
            return cc

        lax.fori_loop(0, _VSTEPS, body, 0)

    def pair_body(g, carry):
        c = g * 2
        issue(c + 1, 1)
        drain(0)
        compute(0)
        issue(c + 2, 0)
        drain(1)
        compute(1)
        return carry

    lax.fori_loop(0, (_NCHUNK - 1) // 2, pair_body, 0)
    drain(0)
    compute(0)

    pltpu.sync_copy(acc_v, out_hbm.at[pl.ds(wid * _NP, _NP)])


@functools.partial(
    pl.kernel,
    out_type=jax.ShapeDtypeStruct((_NP,), jnp.float32),
    mesh=_mesh,
    compiler_params=_cp,
    scratch_types=[
        pltpu.VMEM((_NW * _NS,), jnp.float32),
        pltpu.VMEM((_NS,), jnp.float32),
        pltpu.VMEM((_NS,), jnp.float32),
        pltpu.VMEM((_NS,), jnp.float32),
        pltpu.VMEM((_NS,), jnp.float32),
        pltpu.VMEM((_NS,), jnp.float32),
        pltpu.VMEM((_NS,), jnp.float32),
        pltpu.VMEM((_NS,), jnp.float32),
        pltpu.SemaphoreType.DMA,
        pltpu.SemaphoreType.DMA,
    ],
)
def _sc_membrane_kernel(part_hbm, v_hbm, sp_hbm, dec_hbm, cf_hbm, vth_hbm,
                        nrm_hbm, out_hbm, part_v, v_v, sp_v, dec_v, cf_v,
                        vth_v, nrm_v, out_v, semp, semn):
    cid = lax.axis_index("c")
    sub = lax.axis_index("s")
    wid = cid * 16 + sub
    toff = wid * _NS
    is_last = wid == _NW - 1

    hs = [pltpu.async_copy(part_hbm.at[pl.ds(k * _NP + toff, _NS)],
                           part_v.at[pl.ds(k * _NS, _NS)], semp)
          for k in range(_NW)]

    def issue_neuron(sz):
        return (
            pltpu.async_copy(v_hbm.at[pl.ds(toff, sz)],
                             v_v.at[pl.ds(0, sz)], semn),
            pltpu.async_copy(sp_hbm.at[pl.ds(toff, sz)],
                             sp_v.at[pl.ds(0, sz)], semn),
            pltpu.async_copy(dec_hbm.at[pl.ds(toff, sz)],
                             dec_v.at[pl.ds(0, sz)], semn),
            pltpu.async_copy(cf_hbm.at[pl.ds(toff, sz)],
                             cf_v.at[pl.ds(0, sz)], semn),
            pltpu.async_copy(vth_hbm.at[pl.ds(toff, sz)],
                             vth_v.at[pl.ds(0, sz)], semn),
            pltpu.async_copy(nrm_hbm.at[pl.ds(toff, sz)],
                             nrm_v.at[pl.ds(0, sz)], semn),
        )

    @pl.when(is_last)
    def _():
        for h in issue_neuron(_NS_LAST):
            h.wait()

    @pl.when(jnp.logical_not(is_last))
    def _():
        for h in issue_neuron(_NS):
            h.wait()

    for h in hs:
        h.wait()

    nvregs = jnp.where(is_last, _NS_LAST // 16, _NS // 16)

    def body(j, cc):
        sl = pl.ds(j * 16, 16)
        rec = part_v[pl.ds(j * 16, 16)]
        for k in range(1, _NW):
            rec = rec + part_v[pl.ds(k * _NS + j * 16, 16)]
        sp = sp_v[sl]
        v_reset = v_v[sl] * (1.0 - sp)
        new_v = dec_v[sl] * v_reset + cf_v[sl] * rec
        v_scaled = (new_v - vth_v[sl]) / nrm_v[sl]
        out_v[sl] = jnp.where(v_scaled > 0.0,
                              jnp.float32(1.0), jnp.float32(0.0))
        return cc

    lax.fori_loop(0, nvregs, body, 0)

    @pl.when(is_last)
    def _():
        pltpu.sync_copy(out_v.at[pl.ds(0, _NS_LAST)],
                        out_hbm.at[pl.ds(toff, _NS_LAST)])

    @pl.when(jnp.logical_not(is_last))
    def _():
        pltpu.sync_copy(out_v, out_hbm.at[pl.ds(toff, _NS)])


def kernel(spikes, v, weights, syn_ids, indices, basis, decay,
           current_factor, v_th, normalizer):
    pre = indices[:, 1]
    post = indices[:, 0]
    basis_flat = basis.reshape(_T * _R)
    spikes_flat = spikes.reshape(_N)

    partial = _sc_synapse_kernel(pre, post, syn_ids, weights, spikes_flat,
                                 basis_flat)

    z = _sc_membrane_kernel(partial, v.reshape(_N), spikes_flat, decay,
                            current_factor, v_th, normalizer)
    return z[:_N].reshape(1, _N)
